# baseline (speedup 1.0000x reference)
import functools
import jax
import jax.numpy as jnp
from jax import lax
from jax.experimental import pallas as pl
from jax.experimental.pallas import tpu as pltpu
from jax.experimental.pallas import tpu_sc as plsc

_INP = 128
_OUT = 128
_DEG = 32
_N = 10000
_L = 16

_NC = 2
_NS = 16
_NW = _NC * _NS
_C = 8
_NCHUNK = _N // _C
_ITERS = (_NCHUNK + _NW - 1) // _NW
_EXTRA = _NCHUNK - _NW * (_ITERS - 1)

_mesh = plsc.VectorSubcoreMesh(
    core_axis_name="c", subcore_axis_name="s", num_cores=_NC, num_subcores=_NS)


@functools.partial(
    pl.kernel,
    out_type=jax.ShapeDtypeStruct((_N, _INP), jnp.float32),
    mesh=_mesh,
    scratch_types=[
        pltpu.VMEM((2, _C, _DEG, _INP), jnp.float32),
        pltpu.VMEM((2, _C, _INP), jnp.float32),
        pltpu.SemaphoreType.DMA((2,)),
    ],
)
def _sc_min(mb_hbm, out_hbm, buf, obuf, isems):
    wid = lax.axis_index("s") * _NC + lax.axis_index("c")
    nmine = (_ITERS - 1) + (wid < _EXTRA).astype(jnp.int32)

    def chunk_of(i):
        return jnp.where(i < nmine, wid + _NW * i, wid)

    def in_cp(g, slot):
        return pltpu.make_async_copy(
            mb_hbm.at[pl.ds(g * _C, _C)], buf.at[slot], isems.at[slot])

    in_cp(chunk_of(0), 0).start()

    def outer(k, carry):
        for b in range(2):
            i = 2 * k + b
            g = chunk_of(i)
            in_cp(g, b).wait()

            @pl.when(i + 1 < _ITERS)
            def _():
                in_cp(chunk_of(i + 1), 1 - b).start()

            def node_body(n, c2):
                for cc in range(_INP // _L):
                    sl = pl.ds(cc * _L, _L)
                    acc = buf[b, n, 0, sl]
                    for r in range(1, _DEG):
                        acc = jnp.minimum(acc, buf[b, n, r, sl])
                    obuf[b, n, sl] = acc
                return c2

            lax.fori_loop(0, _C, node_body, 0, unroll=False)
            pltpu.sync_copy(obuf.at[b], out_hbm.at[pl.ds(g * _C, _C)])
        return carry

    lax.fori_loop(0, _ITERS // 2, outer, 0, unroll=False)


def _lin_body(hm_ref, nf_ref, w1_ref, w2_ref, b_ref, out_ref):
    acc = jnp.dot(hm_ref[...], w1_ref[...], preferred_element_type=jnp.float32)
    acc = acc + jnp.dot(nf_ref[...], w2_ref[...], preferred_element_type=jnp.float32)
    out_ref[...] = acc + b_ref[...]


_BN = 2000


def kernel(mailbox_h, node_feat, W, b):
    W1T = W[:, :_INP].T
    W2T = W[:, _INP:].T
    b2 = b.reshape(1, _OUT)
    h_min = _sc_min(mailbox_h)
    return pl.pallas_call(
        _lin_body,
        grid=(_N // _BN,),
        in_specs=[
            pl.BlockSpec((_BN, _INP), lambda i: (i, 0)),
            pl.BlockSpec((_BN, _INP), lambda i: (i, 0)),
            pl.BlockSpec((_INP, _OUT), lambda i: (0, 0)),
            pl.BlockSpec((_INP, _OUT), lambda i: (0, 0)),
            pl.BlockSpec((1, _OUT), lambda i: (0, 0)),
        ],
        out_specs=pl.BlockSpec((_BN, _OUT), lambda i: (i, 0)),
        out_shape=jax.ShapeDtypeStruct((_N, _OUT), jnp.float32),
    )(h_min, node_feat, W1T, W2T, b2)
